# baseline (speedup 1.0000x reference)
import functools

import jax
import jax.numpy as jnp
from jax import lax
from jax.experimental import pallas as pl
from jax.experimental.pallas import tpu as pltpu, tpu_sc as plsc

N = 10000
E = 160000
D = 16
ED = 16
KD = 64
L2 = D * D

NC = 2
NS = 16
NW = NC * NS
CH = 128
EPW = 5120
NCH = EPW // CH
E_PAD = NW * EPW
NSP = 10240
ROWS_PER_SUB = NSP // NS


@functools.cache
def _sc_gather_kernel():
    mesh = plsc.VectorSubcoreMesh(core_axis_name="c", subcore_axis_name="s")
    return functools.partial(
        pl.kernel,
        out_type=jax.ShapeDtypeStruct((E_PAD, D), jnp.float32),
        mesh=mesh,
        scratch_types=[
            pltpu.VMEM((NCH, CH), jnp.int32),
            pltpu.VMEM((EPW, D), jnp.float32),
            pltpu.VMEM_SHARED((N, D), jnp.float32),
            pltpu.SemaphoreType.DMA,
        ],
        compiler_params=pltpu.CompilerParams(use_tc_tiling_on_sc=False),
    )(_sc_gather_body)


def _sc_gather_body(x_hbm, src_hbm, out_hbm, idx_v, rows_v, x_shared, sem):
    cid = lax.axis_index("c")
    sid = lax.axis_index("s")
    wid = sid * NC + cid
    nrows = N // NS
    pltpu.sync_copy(
        x_hbm.at[pl.ds(sid * nrows, nrows)], x_shared.at[pl.ds(sid * nrows, nrows)]
    )
    pltpu.sync_copy(src_hbm.at[wid], idx_v)
    plsc.subcore_barrier()

    def chunk_group(g, carry):
        handles = []
        for b in range(8):
            j = g * 8 + b
            handles.append(
                pltpu.async_copy(
                    x_shared.at[idx_v.at[j]], rows_v.at[pl.ds(j * CH, CH)], sem
                )
            )
        for h in handles:
            h.wait()
        return carry

    lax.fori_loop(0, NCH // 8, chunk_group, 0)
    pltpu.sync_copy(rows_v, out_hbm.at[pl.ds(wid * EPW, EPW)])


@functools.cache
def _sc_scatter_kernel():
    mesh = plsc.VectorSubcoreMesh(core_axis_name="c", subcore_axis_name="s")
    return functools.partial(
        pl.kernel,
        out_type=jax.ShapeDtypeStruct((NC, NSP, D), jnp.float32),
        mesh=mesh,
        scratch_types=[
            pltpu.VMEM((NCH, CH), jnp.int32),
            pltpu.VMEM((EPW, D), jnp.float32),
            pltpu.VMEM((ROWS_PER_SUB, D), jnp.float32),
            pltpu.VMEM_SHARED((NSP, D), jnp.float32),
            pltpu.SemaphoreType.DMA,
        ],
        compiler_params=pltpu.CompilerParams(use_tc_tiling_on_sc=False),
    )(_sc_scatter_body)


def _sc_scatter_body(msg_hbm, dst_hbm, out_hbm, idx_v, msg_v, buf_v, acc_shared, sem):
    cid = lax.axis_index("c")
    sid = lax.axis_index("s")
    wid = sid * NC + cid

    zrow = jnp.zeros((D,), jnp.float32)

    def zbody(i, carry):
        buf_v[i, :] = zrow
        return carry

    lax.fori_loop(0, ROWS_PER_SUB, zbody, 0)
    pltpu.sync_copy(buf_v, acc_shared.at[pl.ds(sid * ROWS_PER_SUB, ROWS_PER_SUB)])
    plsc.subcore_barrier()

    pltpu.sync_copy(dst_hbm.at[wid], idx_v)
    pltpu.sync_copy(msg_hbm.at[pl.ds(wid * EPW, EPW)], msg_v)

    def chunk_group(g, carry):
        handles = []
        for b in range(8):
            j = g * 8 + b
            handles.append(
                pltpu.async_copy(
                    msg_v.at[pl.ds(j * CH, CH)],
                    acc_shared.at[idx_v.at[j]],
                    sem,
                    add=True,
                )
            )
        for h in handles:
            h.wait()
        return carry

    lax.fori_loop(0, NCH // 8, chunk_group, 0)
    plsc.subcore_barrier()

    pltpu.sync_copy(acc_shared.at[pl.ds(sid * ROWS_PER_SUB, ROWS_PER_SUB)], buf_v)
    pltpu.sync_copy(buf_v, out_hbm.at[cid, pl.ds(sid * ROWS_PER_SUB, ROWS_PER_SUB)])


_TE = 1280


def _tc_msg_body(ea, xj, kw1, kb1, kw2, kb2, kw3, kb3, r, s, out):
    bf = jnp.bfloat16
    h = jnp.dot(ea[...].astype(bf), kw1[...].astype(bf),
                preferred_element_type=jnp.float32) + kb1[...]
    h = jnp.maximum(h, 0.0)
    h = jnp.dot(h.astype(bf), kw2[...].astype(bf),
                preferred_element_type=jnp.float32) + kb2[...]
    h = jnp.maximum(h, 0.0)
    w = jnp.dot(h.astype(bf), kw3[...].astype(bf),
                preferred_element_type=jnp.float32) + kb3[...]
    xe = jnp.dot(xj[...].astype(bf), r[...].astype(bf),
                 preferred_element_type=jnp.float32)
    out[...] = jnp.dot((xe * w).astype(bf), s[...].astype(bf),
                       preferred_element_type=jnp.float32)


def _tc_msg(ea, xj, kw1, kb1, kw2, kb2, kw3, kb3, r, s):
    grid = E // _TE
    full = lambda shape: pl.BlockSpec(shape, lambda i: (0, 0))
    return pl.pallas_call(
        _tc_msg_body,
        grid=grid,
        in_specs=[
            pl.BlockSpec((_TE, ED), lambda i: (i, 0)),
            pl.BlockSpec((_TE, D), lambda i: (i, 0)),
            full((ED, KD)),
            full((1, KD)),
            full((KD, KD)),
            full((1, KD)),
            full((KD, L2)),
            full((1, L2)),
            full((D, L2)),
            full((L2, D)),
        ],
        out_specs=pl.BlockSpec((_TE, D), lambda i: (i, 0)),
        out_shape=jax.ShapeDtypeStruct((E_PAD, D), jnp.float32),
        compiler_params=pltpu.CompilerParams(
            dimension_semantics=("arbitrary",),
        ),
    )(ea, xj, kw1, kb1, kw2, kb2, kw3, kb3, r, s)


def _tc_update_body(p, x, root, bias, out, *, apply_gelu):
    y = (
        p[0]
        + p[1]
        + jnp.dot(x[...], root[...], preferred_element_type=jnp.float32)
        + bias[...]
    )
    if apply_gelu:
        y = 0.5 * y * (1.0 + lax.erf(y * 0.7071067811865476))
    out[...] = y


def _tc_update(parts, x, root, bias, apply_gelu):
    return pl.pallas_call(
        functools.partial(_tc_update_body, apply_gelu=apply_gelu),
        grid=1,
        in_specs=[
            pl.BlockSpec((NC, N, D), lambda i: (0, 0, 0)),
            pl.BlockSpec((N, D), lambda i: (0, 0)),
            pl.BlockSpec((D, D), lambda i: (0, 0)),
            pl.BlockSpec((1, D), lambda i: (0, 0)),
        ],
        out_specs=pl.BlockSpec((N, D), lambda i: (0, 0)),
        out_shape=jax.ShapeDtypeStruct((N, D), jnp.float32),
    )(parts, x, root, bias)


def kernel(nodes, edge_index, edge_attr, KW1, Kb1, KW2, Kb2, KW3, Kb3,
           root0, bias0, root1, bias1):
    src = edge_index[0]
    dst = edge_index[1]
    pad = E_PAD - E
    src_c = jnp.concatenate([src, jnp.zeros((pad,), jnp.int32)]).reshape(NW, NCH, CH)
    dst_c = jnp.concatenate([dst, jnp.full((pad,), N, jnp.int32)]).reshape(NW, NCH, CH)

    m = jnp.arange(L2)
    r_mat = (jnp.arange(D)[:, None] == (m // D)[None, :]).astype(jnp.float32)
    s_mat = ((m % D)[:, None] == jnp.arange(D)[None, :]).astype(jnp.float32)

    kb1 = Kb1.reshape(1, KD)
    kb2 = Kb2.reshape(1, KD)
    kb3 = Kb3.reshape(1, L2)
    b0 = bias0.reshape(1, D)
    b1 = bias1.reshape(1, D)

    x = nodes
    for root, bias, gelu in ((root0, b0, True), (root1, b1, False)):
        xj = _sc_gather_kernel()(x, src_c)
        msg = _tc_msg(edge_attr, xj, KW1, kb1, KW2, kb2, KW3, kb3, r_mat, s_mat)
        parts = _sc_scatter_kernel()(msg, dst_c)
        x = _tc_update(parts, x, root, bias, gelu)
    return x


# speedup vs baseline: 1.4453x; 1.4453x over previous
import functools

import jax
import jax.numpy as jnp
from jax import lax
from jax.experimental import pallas as pl
from jax.experimental.pallas import tpu as pltpu, tpu_sc as plsc

N = 10000
E = 160000
D = 16
ED = 16
KD = 64
L2 = D * D

NC = 2
NS = 16
NW = NC * NS
CH = 128
EPW = 5120
NCH = EPW // CH
E_PAD = NW * EPW
NSP = 10240
ROWS_PER_SUB = NSP // NS


@functools.cache
def _sc_gather_kernel():
    mesh = plsc.VectorSubcoreMesh(core_axis_name="c", subcore_axis_name="s")
    return functools.partial(
        pl.kernel,
        out_type=jax.ShapeDtypeStruct((E_PAD, D), jnp.float32),
        mesh=mesh,
        scratch_types=[
            pltpu.VMEM((NCH, CH), jnp.int32),
            pltpu.VMEM((EPW, D), jnp.float32),
            pltpu.VMEM_SHARED((N, D), jnp.float32),
            pltpu.SemaphoreType.DMA,
        ],
        compiler_params=pltpu.CompilerParams(use_tc_tiling_on_sc=False),
    )(_sc_gather_body)


def _sc_gather_body(x_hbm, src_hbm, out_hbm, idx_v, rows_v, x_shared, sem):
    cid = lax.axis_index("c")
    sid = lax.axis_index("s")
    wid = sid * NC + cid
    nrows = N // NS
    pltpu.sync_copy(
        x_hbm.at[pl.ds(sid * nrows, nrows)], x_shared.at[pl.ds(sid * nrows, nrows)]
    )
    pltpu.sync_copy(src_hbm.at[wid], idx_v)
    plsc.subcore_barrier()

    def chunk_group(g, carry):
        handles = []
        for b in range(8):
            j = g * 8 + b
            handles.append(
                pltpu.async_copy(
                    x_shared.at[idx_v.at[j]], rows_v.at[pl.ds(j * CH, CH)], sem
                )
            )
        for h in handles:
            h.wait()
        return carry

    lax.fori_loop(0, NCH // 8, chunk_group, 0)
    pltpu.sync_copy(rows_v, out_hbm.at[pl.ds(wid * EPW, EPW)])


@functools.cache
def _sc_scatter_kernel():
    mesh = plsc.VectorSubcoreMesh(core_axis_name="c", subcore_axis_name="s")
    return functools.partial(
        pl.kernel,
        out_type=jax.ShapeDtypeStruct((NC, NSP, D), jnp.float32),
        mesh=mesh,
        scratch_types=[
            pltpu.VMEM((NCH, CH), jnp.int32),
            pltpu.VMEM((EPW, D), jnp.float32),
            pltpu.VMEM((ROWS_PER_SUB, D), jnp.float32),
            pltpu.VMEM_SHARED((NSP, D), jnp.float32),
            pltpu.SemaphoreType.DMA,
        ],
        compiler_params=pltpu.CompilerParams(use_tc_tiling_on_sc=False),
    )(_sc_scatter_body)


def _sc_scatter_body(msg_hbm, dst_hbm, out_hbm, idx_v, msg_v, buf_v, acc_shared, sem):
    cid = lax.axis_index("c")
    sid = lax.axis_index("s")
    wid = sid * NC + cid

    zrow = jnp.zeros((D,), jnp.float32)

    def zbody(i, carry):
        buf_v[i, :] = zrow
        return carry

    lax.fori_loop(0, ROWS_PER_SUB, zbody, 0)
    pltpu.sync_copy(buf_v, acc_shared.at[pl.ds(sid * ROWS_PER_SUB, ROWS_PER_SUB)])
    plsc.subcore_barrier()

    pltpu.sync_copy(dst_hbm.at[wid], idx_v)
    pltpu.sync_copy(msg_hbm.at[pl.ds(wid * EPW, EPW)], msg_v)

    def chunk_group(g, carry):
        handles = []
        for b in range(8):
            j = g * 8 + b
            handles.append(
                pltpu.async_copy(
                    msg_v.at[pl.ds(j * CH, CH)],
                    acc_shared.at[idx_v.at[j]],
                    sem,
                    add=True,
                )
            )
        for h in handles:
            h.wait()
        return carry

    lax.fori_loop(0, NCH // 8, chunk_group, 0)
    plsc.subcore_barrier()

    pltpu.sync_copy(acc_shared.at[pl.ds(sid * ROWS_PER_SUB, ROWS_PER_SUB)], buf_v)
    pltpu.sync_copy(buf_v, out_hbm.at[cid, pl.ds(sid * ROWS_PER_SUB, ROWS_PER_SUB)])


_TE = 1280
_PK = 8
_TR = _TE // _PK
_ER = E * D // 128
_ER_PAD = E_PAD * D // 128


def _tc_msg_body(ea, xj, k1, b1, k2, b2, k3, b3, r, s, out):
    bf = jnp.bfloat16
    h = jnp.dot(ea[...].astype(bf), k1[...], preferred_element_type=jnp.float32)
    h = jnp.maximum(h + b1[...], 0.0)
    h = jnp.dot(h.astype(bf), k2[...], preferred_element_type=jnp.float32)
    h = jnp.maximum(h + b2[...], 0.0)
    w = jnp.dot(h.astype(bf), k3[...], preferred_element_type=jnp.float32) + b3[...]
    xe = jnp.dot(xj[...].astype(bf), r[...], preferred_element_type=jnp.float32)
    out[...] = jnp.dot((xe * w).astype(bf), s[...],
                       preferred_element_type=jnp.float32)


def _tc_msg(ea_p, xj_p, k1, b1, k2, b2, k3, b3, r, s):
    grid = E // _TE
    full = lambda shape: pl.BlockSpec(shape, lambda i: (0, 0))
    return pl.pallas_call(
        _tc_msg_body,
        grid=grid,
        in_specs=[
            pl.BlockSpec((_TR, 128), lambda i: (i, 0)),
            pl.BlockSpec((_TR, 128), lambda i: (i, 0)),
            full((_PK * ED, _PK * KD)),
            full((1, _PK * KD)),
            full((_PK * KD, _PK * KD)),
            full((1, _PK * KD)),
            full((_PK * KD, _PK * L2)),
            full((1, _PK * L2)),
            full((_PK * D, _PK * L2)),
            full((_PK * L2, _PK * D)),
        ],
        out_specs=pl.BlockSpec((_TR, 128), lambda i: (i, 0)),
        out_shape=jax.ShapeDtypeStruct((_ER_PAD, 128), jnp.float32),
        compiler_params=pltpu.CompilerParams(
            dimension_semantics=("arbitrary",),
        ),
    )(ea_p, xj_p, k1, b1, k2, b2, k3, b3, r, s)


def _tc_update_body(p, x, root, bias, out, *, apply_gelu):
    y = (
        p[0]
        + p[1]
        + jnp.dot(x[...], root[...], preferred_element_type=jnp.float32)
        + bias[...]
    )
    if apply_gelu:
        y = 0.5 * y * (1.0 + lax.erf(y * 0.7071067811865476))
    out[...] = y


def _tc_update(parts, x, root, bias, apply_gelu):
    return pl.pallas_call(
        functools.partial(_tc_update_body, apply_gelu=apply_gelu),
        grid=1,
        in_specs=[
            pl.BlockSpec((NC, N, D), lambda i: (0, 0, 0)),
            pl.BlockSpec((N, D), lambda i: (0, 0)),
            pl.BlockSpec((D, D), lambda i: (0, 0)),
            pl.BlockSpec((1, D), lambda i: (0, 0)),
        ],
        out_specs=pl.BlockSpec((N, D), lambda i: (0, 0)),
        out_shape=jax.ShapeDtypeStruct((N, D), jnp.float32),
    )(parts, x, root, bias)


def kernel(nodes, edge_index, edge_attr, KW1, Kb1, KW2, Kb2, KW3, Kb3,
           root0, bias0, root1, bias1):
    src = edge_index[0]
    dst = edge_index[1]
    pad = E_PAD - E
    src_c = jnp.concatenate([src, jnp.zeros((pad,), jnp.int32)]).reshape(NW, NCH, CH)
    dst_c = jnp.concatenate([dst, jnp.full((pad,), N, jnp.int32)]).reshape(NW, NCH, CH)

    m = jnp.arange(L2)
    r_mat = (jnp.arange(D)[:, None] == (m // D)[None, :]).astype(jnp.float32)
    s_mat = ((m % D)[:, None] == jnp.arange(D)[None, :]).astype(jnp.float32)

    bf = jnp.bfloat16
    eye8 = jnp.eye(_PK, dtype=jnp.float32)
    k1 = jnp.kron(eye8, KW1).astype(bf)
    k2 = jnp.kron(eye8, KW2).astype(bf)
    k3 = jnp.kron(eye8, KW3).astype(bf)
    r_p = jnp.kron(eye8, r_mat).astype(bf)
    s_p = jnp.kron(eye8, s_mat).astype(bf)
    b1_p = jnp.tile(Kb1, _PK).reshape(1, _PK * KD)
    b2_p = jnp.tile(Kb2, _PK).reshape(1, _PK * KD)
    b3_p = jnp.tile(Kb3, _PK).reshape(1, _PK * L2)
    b0 = bias0.reshape(1, D)
    b1 = bias1.reshape(1, D)

    ea_p = edge_attr.reshape(_ER, 128)

    x = nodes
    for root, bias, gelu in ((root0, b0, True), (root1, b1, False)):
        xj = _sc_gather_kernel()(x, src_c)
        msg_p = _tc_msg(ea_p, xj.reshape(_ER_PAD, 128),
                        k1, b1_p, k2, b2_p, k3, b3_p, r_p, s_p)
        parts = _sc_scatter_kernel()(msg_p.reshape(E_PAD, D), dst_c)
        x = _tc_update(parts, x, root, bias, gelu)
    return x
